# baseline (speedup 1.0000x reference)
import functools

import jax
import jax.numpy as jnp
from jax import lax
from jax.experimental import pallas as pl
from jax.experimental.pallas import tpu as pltpu
from jax.experimental.pallas import tpu_sc as plsc

D = 64
L_SEQ = 200
B = 1024
NUM_CORES = 2
NUM_SUBCORES = 16
LANES = 16
QTOK = 256
NQ = B // QTOK


@functools.lru_cache(maxsize=None)
def _build_call():
    mesh = plsc.VectorSubcoreMesh(core_axis_name="c", subcore_axis_name="s")

    @functools.partial(
        pl.kernel,
        mesh=mesh,
        out_type=jax.ShapeDtypeStruct((L_SEQ, 8, 8, 8, 128), jnp.float32),
        compiler_params=pltpu.CompilerParams(
            use_tc_tiling_on_sc=False, needs_layout_passes=False
        ),
        scratch_types=[
            pltpu.VMEM((8, 128), jnp.int32),
            pltpu.VMEM((B,), jnp.int32),
            pltpu.VMEM((B,), jnp.int32),
            pltpu.VMEM((128,), jnp.float32),
            pltpu.VMEM((QTOK, 128), jnp.float32),
            pltpu.VMEM((QTOK, 128), jnp.float32),
            pltpu.VMEM((8, 2, 8, 128), jnp.float32),
            pltpu.VMEM((8, 2, 8, 128), jnp.float32),
            pltpu.SemaphoreType.DMA,
            pltpu.SemaphoreType.DMA,
            pltpu.SemaphoreType.DMA,
        ],
    )
    def emb(ids5, tbl2, pos2, out5, ids_raw, idx2_v, par64_v, pos_row,
            g0, g1, s0, s1, isem, gsem, osem):
        wid = lax.axis_index("s") * NUM_CORES + lax.axis_index("c")
        n_l = jnp.where(wid < 8, 7, 6)
        gbufs = (g0, g1)
        sbufs = (s0, s1)
        iota16 = lax.iota(jnp.int32, LANES)
        rvs = [bg * LANES + iota16 for bg in range(16)]

        def l_body(k, carry):
            l = wid + 32 * k
            tl = l // 8
            sl = l % 8

            pdesc = pltpu.async_copy(pos2.at[l // 2], pos_row, isem)
            idescs = [
                pltpu.async_copy(ids5.at[tl, tb, sl], ids_raw.at[tb], isem)
                for tb in range(8)
            ]
            pdesc.wait()
            for dsc in idescs:
                dsc.wait()

            def t_body(i, c):
                v = ids_raw[i // 8, pl.ds((i % 8) * LANES, LANES)]
                idx2_v[pl.ds(i * LANES, LANES)] = lax.shift_right_logical(v, 1)
                par64_v[pl.ds(i * LANES, LANES)] = lax.shift_left(
                    lax.bitwise_and(v, 1), 6
                )
                return c

            lax.fori_loop(0, B // LANES, t_body, 0)

            def start_gather(q):
                return pltpu.async_copy(
                    tbl2.at[idx2_v.at[pl.ds(q * QTOK, QTOK)]],
                    gbufs[q % 2],
                    gsem,
                )

            gdesc = start_gather(0)
            sdescs = [None, None]
            for q in range(NQ):
                gbuf = gbufs[q % 2]
                sbuf = sbufs[q % 2]
                gdesc.wait()
                if q + 1 < NQ:
                    gdesc = start_gather(q + 1)
                if sdescs[q % 2] is not None:
                    sdescs[q % 2].wait()
                    sdescs[q % 2] = None

                poff = (l % 2) * D
                pvs = [
                    par64_v[pl.ds(q * QTOK + bg * LANES, LANES)]
                    for bg in range(16)
                ]

                def d_body(dd, c):
                    ti = dd // 8
                    s2 = dd % 8
                    dsplat = jnp.full((LANES,), dd, jnp.int32)
                    ps = plsc.load_gather(
                        pos_row, [jnp.full((LANES,), poff + dd, jnp.int32)]
                    )
                    for bg in range(16):
                        cv = pvs[bg] + dsplat
                        g = plsc.load_gather(gbuf, [rvs[bg], cv])
                        sbuf[ti, bg // 8, s2, pl.ds((bg % 8) * LANES, LANES)] = (
                            g + ps
                        )
                    return c

                lax.fori_loop(0, D, d_body, 0)

                sdescs[q % 2] = pltpu.async_copy(
                    sbuf, out5.at[l, :, pl.ds(2 * q, 2)], osem
                )
            for dsc in sdescs:
                if dsc is not None:
                    dsc.wait()
            return carry

        lax.fori_loop(0, n_l, l_body, 0)

    return emb


def kernel(input_ids, token_table, position_table):
    ids5 = input_ids.astype(jnp.int32).reshape(8, 128, 25, 8).transpose(2, 0, 3, 1)
    tbl2 = token_table.reshape(500000, 128)
    pos2 = position_table.reshape(100, 128)
    out5 = _build_call()(ids5, tbl2, pos2)
    return out5.transpose(2, 4, 0, 1, 3).reshape(B, L_SEQ, D)


# speedup vs baseline: 1.2262x; 1.2262x over previous
import functools

import jax
import jax.numpy as jnp
from jax import lax
from jax.experimental import pallas as pl
from jax.experimental.pallas import tpu as pltpu
from jax.experimental.pallas import tpu_sc as plsc

D = 64
L_SEQ = 200
B = 1024
NUM_CORES = 2
NUM_SUBCORES = 16
LANES = 16
QTOK = 256
NQ = B // QTOK


@functools.lru_cache(maxsize=None)
def _build_call():
    mesh = plsc.VectorSubcoreMesh(core_axis_name="c", subcore_axis_name="s")

    @functools.partial(
        pl.kernel,
        mesh=mesh,
        out_type=jax.ShapeDtypeStruct((L_SEQ, 8, 8, 8, 128), jnp.float32),
        compiler_params=pltpu.CompilerParams(
            use_tc_tiling_on_sc=False, needs_layout_passes=False
        ),
        scratch_types=[
            pltpu.VMEM((8, 128), jnp.int32),
            pltpu.VMEM((B,), jnp.int32),
            pltpu.VMEM((B,), jnp.int32),
            pltpu.VMEM((128,), jnp.float32),
            pltpu.VMEM((QTOK, 128), jnp.float32),
            pltpu.VMEM((QTOK, 128), jnp.float32),
            pltpu.VMEM((8, 2, 8, 128), jnp.float32),
            pltpu.VMEM((8, 2, 8, 128), jnp.float32),
            pltpu.SemaphoreType.DMA,
            pltpu.SemaphoreType.DMA,
            pltpu.SemaphoreType.DMA,
        ],
    )
    def emb(ids5, tbl2, pos2, out5, ids_raw, idx2_v, par64_v, pos_row,
            g0, g1, s0, s1, isem, gsem, osem):
        wid = lax.axis_index("s") * NUM_CORES + lax.axis_index("c")
        n_l = jnp.where(wid < 8, 7, 6)
        gbufs = (g0, g1)
        sbufs = (s0, s1)
        iota16 = lax.iota(jnp.int32, LANES)
        rvs = [bg * LANES + iota16 for bg in range(16)]

        def l_body(k, carry):
            l = wid + 32 * k
            tl = l // 8
            sl = l % 8

            pdesc = pltpu.async_copy(pos2.at[l // 2], pos_row, isem)
            idescs = [
                pltpu.async_copy(ids5.at[tl, tb, sl], ids_raw.at[tb], isem)
                for tb in range(8)
            ]
            pdesc.wait()
            for dsc in idescs:
                dsc.wait()

            @plsc.parallel_loop(0, B // LANES)
            def t_body(i):
                v = ids_raw[i // 8, pl.ds((i % 8) * LANES, LANES)]
                idx2_v[pl.ds(i * LANES, LANES)] = lax.shift_right_logical(v, 1)
                par64_v[pl.ds(i * LANES, LANES)] = lax.shift_left(
                    lax.bitwise_and(v, 1), 6
                )

            def start_gather(q):
                return pltpu.async_copy(
                    tbl2.at[idx2_v.at[pl.ds(q * QTOK, QTOK)]],
                    gbufs[q % 2],
                    gsem,
                )

            gdesc = start_gather(0)
            sdescs = [None, None]
            for q in range(NQ):
                gbuf = gbufs[q % 2]
                sbuf = sbufs[q % 2]
                gdesc.wait()
                if q + 1 < NQ:
                    gdesc = start_gather(q + 1)
                if sdescs[q % 2] is not None:
                    sdescs[q % 2].wait()
                    sdescs[q % 2] = None

                poff = (l % 2) * D
                pvs = [
                    par64_v[pl.ds(q * QTOK + bg * LANES, LANES)]
                    for bg in range(16)
                ]

                @plsc.parallel_loop(0, D)
                def d_body(dd):
                    ti = dd // 8
                    s2 = dd % 8
                    dsplat = jnp.full((LANES,), dd, jnp.int32)
                    ps = plsc.load_gather(
                        pos_row, [jnp.full((LANES,), poff + dd, jnp.int32)]
                    )
                    for bg in range(16):
                        cv = pvs[bg] + dsplat
                        g = plsc.load_gather(gbuf, [rvs[bg], cv])
                        sbuf[ti, bg // 8, s2, pl.ds((bg % 8) * LANES, LANES)] = (
                            g + ps
                        )

                sdescs[q % 2] = pltpu.async_copy(
                    sbuf, out5.at[l, :, pl.ds(2 * q, 2)], osem
                )
            for dsc in sdescs:
                if dsc is not None:
                    dsc.wait()
            return carry

        lax.fori_loop(0, n_l, l_body, 0)

    return emb


def kernel(input_ids, token_table, position_table):
    ids5 = input_ids.astype(jnp.int32).reshape(8, 128, 25, 8).transpose(2, 0, 3, 1)
    tbl2 = token_table.reshape(500000, 128)
    pos2 = position_table.reshape(100, 128)
    out5 = _build_call()(ids5, tbl2, pos2)
    return out5.transpose(2, 4, 0, 1, 3).reshape(B, L_SEQ, D)


# speedup vs baseline: 1.2296x; 1.0028x over previous
import functools

import jax
import jax.numpy as jnp
from jax import lax
from jax.experimental import pallas as pl
from jax.experimental.pallas import tpu as pltpu
from jax.experimental.pallas import tpu_sc as plsc

D = 64
L_SEQ = 200
B = 1024
NUM_CORES = 2
NUM_SUBCORES = 16
LANES = 16
QTOK = 256
NQ = B // QTOK


@functools.lru_cache(maxsize=None)
def _build_call():
    mesh = plsc.VectorSubcoreMesh(core_axis_name="c", subcore_axis_name="s")

    @functools.partial(
        pl.kernel,
        mesh=mesh,
        out_type=jax.ShapeDtypeStruct((L_SEQ, 8, 8, 8, 128), jnp.float32),
        compiler_params=pltpu.CompilerParams(
            use_tc_tiling_on_sc=False, needs_layout_passes=False
        ),
        scratch_types=[
            pltpu.VMEM((8, 128), jnp.int32),
            pltpu.VMEM((B,), jnp.int32),
            pltpu.VMEM((B,), jnp.int32),
            pltpu.VMEM((128,), jnp.float32),
            pltpu.VMEM((QTOK, 128), jnp.float32),
            pltpu.VMEM((QTOK, 128), jnp.float32),
            pltpu.VMEM((8, 2, 8, 128), jnp.float32),
            pltpu.VMEM((8, 2, 8, 128), jnp.float32),
            pltpu.SemaphoreType.DMA,
            pltpu.SemaphoreType.DMA,
            pltpu.SemaphoreType.DMA,
        ],
    )
    def emb(ids5, tbl2, pos2, out5, ids_raw, idx2_v, par64_v, pos_row,
            g0, g1, s0, s1, isem, gsem, osem):
        wid = lax.axis_index("s") * NUM_CORES + lax.axis_index("c")
        n_l = jnp.where(wid < 8, 7, 6)
        gbufs = (g0, g1)
        sbufs = (s0, s1)
        iota16 = lax.iota(jnp.int32, LANES)
        rvs = [bg * LANES + iota16 for bg in range(16)]

        def l_body(k, carry):
            l = wid + 32 * k
            tl = l // 8
            sl = l % 8

            pdesc = pltpu.async_copy(pos2.at[l // 2], pos_row, isem)
            idescs = [
                pltpu.async_copy(ids5.at[tl, tb, sl], ids_raw.at[tb], isem)
                for tb in range(8)
            ]
            pdesc.wait()
            for dsc in idescs:
                dsc.wait()

            @plsc.parallel_loop(0, B // LANES)
            def t_body(i):
                v = ids_raw[i // 8, pl.ds((i % 8) * LANES, LANES)]
                idx2_v[pl.ds(i * LANES, LANES)] = lax.shift_right_logical(v, 1)
                par64_v[pl.ds(i * LANES, LANES)] = lax.shift_left(
                    lax.bitwise_and(v, 1), 6
                )

            def start_gather(q):
                return pltpu.async_copy(
                    tbl2.at[idx2_v.at[pl.ds(q * QTOK, QTOK)]],
                    gbufs[q % 2],
                    gsem,
                )

            gdesc = start_gather(0)
            sdescs = [None, None]
            for q in range(NQ):
                gbuf = gbufs[q % 2]
                sbuf = sbufs[q % 2]
                gdesc.wait()
                if q + 1 < NQ:
                    gdesc = start_gather(q + 1)
                if sdescs[q % 2] is not None:
                    sdescs[q % 2].wait()
                    sdescs[q % 2] = None

                poff = (l % 2) * D
                pvs = [
                    par64_v[pl.ds(q * QTOK + bg * LANES, LANES)]
                    for bg in range(16)
                ]

                @plsc.parallel_loop(0, D, unroll=4)
                def d_body(dd):
                    ti = dd // 8
                    s2 = dd % 8
                    dsplat = jnp.full((LANES,), dd, jnp.int32)
                    ps = plsc.load_gather(
                        pos_row, [jnp.full((LANES,), poff + dd, jnp.int32)]
                    )
                    for bg in range(16):
                        cv = pvs[bg] + dsplat
                        g = plsc.load_gather(gbuf, [rvs[bg], cv])
                        sbuf[ti, bg // 8, s2, pl.ds((bg % 8) * LANES, LANES)] = (
                            g + ps
                        )

                sdescs[q % 2] = pltpu.async_copy(
                    sbuf, out5.at[l, :, pl.ds(2 * q, 2)], osem
                )
            for dsc in sdescs:
                if dsc is not None:
                    dsc.wait()
            return carry

        lax.fori_loop(0, n_l, l_body, 0)

    return emb


def kernel(input_ids, token_table, position_table):
    ids5 = input_ids.astype(jnp.int32).reshape(8, 128, 25, 8).transpose(2, 0, 3, 1)
    tbl2 = token_table.reshape(500000, 128)
    pos2 = position_table.reshape(100, 128)
    out5 = _build_call()(ids5, tbl2, pos2)
    return out5.transpose(2, 4, 0, 1, 3).reshape(B, L_SEQ, D)


# speedup vs baseline: 1.2994x; 1.0568x over previous
import functools

import jax
import jax.numpy as jnp
from jax import lax
from jax.experimental import pallas as pl
from jax.experimental.pallas import tpu as pltpu
from jax.experimental.pallas import tpu_sc as plsc

D = 64
L_SEQ = 200
NUM_CORES = 2
NUM_SUBCORES = 16
NUM_WORKERS = NUM_CORES * NUM_SUBCORES
LANES = 16

SEQS_PER_CHUNK = 4
CHUNK_ROWS = SEQS_PER_CHUNK * L_SEQ


@functools.lru_cache(maxsize=None)
def _build_call(n_rows: int):
    rows_per_w = n_rows // NUM_WORKERS
    n_chunks = rows_per_w // CHUNK_ROWS
    assert rows_per_w % CHUNK_ROWS == 0

    mesh = plsc.VectorSubcoreMesh(core_axis_name="c", subcore_axis_name="s")

    @functools.partial(
        pl.kernel,
        mesh=mesh,
        out_type=jax.ShapeDtypeStruct((n_rows, D), jnp.float32),
        compiler_params=pltpu.CompilerParams(
            use_tc_tiling_on_sc=False, skip_device_barrier=True
        ),
        scratch_types=[
            pltpu.VMEM((rows_per_w,), jnp.int32),
            pltpu.VMEM((CHUNK_ROWS, D), jnp.float32),
            pltpu.VMEM((CHUNK_ROWS, D), jnp.float32),
            pltpu.VMEM((L_SEQ, D), jnp.float32),
            pltpu.SemaphoreType.DMA,
            pltpu.SemaphoreType.DMA,
        ],
    )
    def emb(ids_hbm, table_hbm, pos_hbm, out_hbm, idx_v, buf0, buf1, pos_v,
            gsem, osem):
        bufs = (buf0, buf1)
        wid = lax.axis_index("s") * NUM_CORES + lax.axis_index("c")
        base = wid * rows_per_w
        pltpu.sync_copy(pos_hbm, pos_v)
        pltpu.sync_copy(ids_hbm.at[pl.ds(base, rows_per_w)], idx_v)

        def start_gather(ck):
            return pltpu.async_copy(
                table_hbm.at[idx_v.at[pl.ds(ck * CHUNK_ROWS, CHUNK_ROWS)]],
                bufs[ck % 2],
                gsem,
            )

        gather_desc = start_gather(0)
        scatter_descs = [None, None]
        for ck in range(n_chunks):
            cur = bufs[ck % 2]
            gather_desc.wait()
            if ck + 1 < n_chunks:
                if scatter_descs[(ck + 1) % 2] is not None:
                    scatter_descs[(ck + 1) % 2].wait()
                    scatter_descs[(ck + 1) % 2] = None
                gather_desc = start_gather(ck + 1)

            @plsc.parallel_loop(0, L_SEQ)
            def _(l):
                pv = [pos_v[l, pl.ds(c * LANES, LANES)] for c in range(D // LANES)]
                for s in range(SEQS_PER_CHUNK):
                    r = s * L_SEQ + l
                    for c in range(D // LANES):
                        sl = pl.ds(c * LANES, LANES)
                        cur[r, sl] = cur[r, sl] + pv[c]

            scatter_descs[ck % 2] = pltpu.async_copy(
                cur, out_hbm.at[pl.ds(base + ck * CHUNK_ROWS, CHUNK_ROWS)], osem
            )
        for d in scatter_descs:
            if d is not None:
                d.wait()

    return emb


def kernel(input_ids, token_table, position_table):
    b, l = input_ids.shape
    ids_flat = input_ids.reshape(b * l).astype(jnp.int32)
    pos = position_table[:l]
    out = _build_call(b * l)(ids_flat, token_table, pos)
    return out.reshape(b, l, D)
